# Initial kernel scaffold; baseline (speedup 1.0000x reference)
import functools

import jax
import jax.numpy as jnp
from jax import lax
from jax.experimental import pallas as pl
from jax.experimental.pallas import tpu as pltpu
from jax.experimental.pallas import tpu_sc as plsc

N = 10000
E = 320000
D = 128

NUM_TILES = 16
EPT = E // NUM_TILES
K = 80
NCH = EPT // K
RPT = N // NUM_TILES
ZR = 125
NZ = RPT // ZR


def _sc_body(feat_hbm, ei_hbm, w_hbm, num_out, den_out,
             num_sh, den_sh, gidx_v, sidx_v, w_v, rows_v, dsrc_v,
             zbuf_v, zbuf16_v, sem):
    c = lax.axis_index("c")
    s = lax.axis_index("s")

    zero16 = jnp.zeros((16,), jnp.float32)

    def zrow(i, carry):
        for g in range(D // 16):
            zbuf_v[i, pl.ds(g * 16, 16)] = zero16
        zbuf16_v[i, pl.ds(0, 16)] = zero16
        return carry

    lax.fori_loop(0, ZR, zrow, 0)
    rbase = s * RPT
    for j in range(NZ):
        pltpu.sync_copy(zbuf_v, num_sh.at[pl.ds(rbase + j * ZR, ZR)])
        pltpu.sync_copy(zbuf16_v, den_sh.at[pl.ds(rbase + j * ZR, ZR)])
    plsc.subcore_barrier()

    ebase = s * EPT

    def echunk(j, carry):
        off = ebase + j * K
        pltpu.sync_copy(ei_hbm.at[c, pl.ds(off, K)], gidx_v)
        pltpu.sync_copy(ei_hbm.at[1 - c, pl.ds(off, K)], sidx_v)
        pltpu.sync_copy(w_hbm.at[pl.ds(off, K)], w_v)
        pltpu.async_copy(feat_hbm.at[gidx_v], rows_v, sem).wait()

        def scale(k, carry2):
            wspl = plsc.load_gather(w_v, [jnp.full((16,), k, jnp.int32)])
            for g in range(D // 16):
                rows_v[k, pl.ds(g * 16, 16)] = rows_v[k, pl.ds(g * 16, 16)] * wspl
            dsrc_v[k, pl.ds(0, 16)] = wspl
            return carry2

        lax.fori_loop(0, K, scale, 0)
        pltpu.sync_copy(rows_v, num_sh.at[sidx_v], add=True)
        pltpu.sync_copy(dsrc_v, den_sh.at[sidx_v], add=True)
        return carry

    lax.fori_loop(0, NCH, echunk, 0)
    plsc.subcore_barrier()

    pltpu.sync_copy(num_sh.at[pl.ds(rbase, RPT)], num_out.at[c, pl.ds(rbase, RPT)])
    pltpu.sync_copy(den_sh.at[pl.ds(rbase, RPT)], den_out.at[c, pl.ds(rbase, RPT)])


def _sc_aggregate(feat, ei, w):
    mesh = plsc.VectorSubcoreMesh(core_axis_name="c", subcore_axis_name="s")
    fn = functools.partial(
        pl.kernel,
        mesh=mesh,
        out_type=[
            jax.ShapeDtypeStruct((2, N, D), jnp.float32),
            jax.ShapeDtypeStruct((2, N, 16), jnp.float32),
        ],
        scratch_types=[
            pltpu.VMEM_SHARED((N, D), jnp.float32),
            pltpu.VMEM_SHARED((N, 16), jnp.float32),
            pltpu.VMEM((K,), jnp.int32),
            pltpu.VMEM((K,), jnp.int32),
            pltpu.VMEM((K,), jnp.float32),
            pltpu.VMEM((K, D), jnp.float32),
            pltpu.VMEM((K, 16), jnp.float32),
            pltpu.VMEM((ZR, D), jnp.float32),
            pltpu.VMEM((ZR, 16), jnp.float32),
            pltpu.SemaphoreType.DMA,
        ],
    )(_sc_body)
    return fn(feat, ei, w)


BN = 1000


def _tc_body(feat, n1m, d1, n2m, d2, w1, w2, wih, whh, bih, bhh, out):
    dot = functools.partial(
        lax.dot_general,
        precision=lax.Precision.HIGHEST,
        preferred_element_type=jnp.float32,
    )
    den1 = d1[:, 0:1]
    den1 = jnp.where(den1 == 0.0, 1.0, den1)
    den2 = d2[:, 0:1]
    den2 = jnp.where(den2 == 0.0, 1.0, den2)
    neigh1 = n1m[...] / den1
    neigh2 = n2m[...] / den2
    n1 = dot(neigh1, w1[...], (((1,), (1,)), ((), ())))
    n2 = dot(neigh2, w2[...], (((1,), (1,)), ((), ())))
    gi = (dot(n1, wih[:, :D], (((1,), (1,)), ((), ())))
          + dot(n2, wih[:, D:], (((1,), (1,)), ((), ())))
          + bih[...])
    gh = dot(feat[...], whh[...], (((1,), (1,)), ((), ()))) + bhh[...]
    i_r, i_z, i_n = gi[:, :D], gi[:, D:2 * D], gi[:, 2 * D:]
    h_r, h_z, h_n = gh[:, :D], gh[:, D:2 * D], gh[:, 2 * D:]
    r = jax.nn.sigmoid(i_r + h_r)
    z = jax.nn.sigmoid(i_z + h_z)
    nn_ = jnp.tanh(i_n + r * h_n)
    out[...] = (1.0 - z) * nn_ + z * feat[...]


def _tc_dense(feat, num1, den1, num2, den2, W1, W2, W_ih, W_hh, b_ih, b_hh):
    grid = (N // BN,)
    row_spec = pl.BlockSpec((BN, D), lambda i: (i, 0))
    den_spec = pl.BlockSpec((BN, 16), lambda i: (i, 0))

    def full(shape):
        return pl.BlockSpec(shape, lambda i: tuple(0 for _ in shape))

    return pl.pallas_call(
        _tc_body,
        grid=grid,
        in_specs=[
            row_spec, row_spec, den_spec, row_spec, den_spec,
            full((D, D)), full((D, D)), full((3 * D, 2 * D)),
            full((3 * D, D)), full((1, 3 * D)), full((1, 3 * D)),
        ],
        out_specs=row_spec,
        out_shape=jax.ShapeDtypeStruct((N, D), jnp.float32),
    )(feat, num1, den1, num2, den2, W1, W2, W_ih, W_hh,
      b_ih.reshape(1, -1), b_hh.reshape(1, -1))


def kernel(feat, edge_index, edge_weight, W1, W2, W_ih, W_hh, b_ih, b_hh):
    ei = edge_index.astype(jnp.int32)
    num, den = _sc_aggregate(feat, ei, edge_weight)
    return _tc_dense(feat, num[0], den[0], num[1], den[1],
                     W1, W2, W_ih, W_hh, b_ih, b_hh)


# speedup vs baseline: 4.0210x; 4.0210x over previous
import functools

import jax
import jax.numpy as jnp
from jax import lax
from jax.experimental import pallas as pl
from jax.experimental.pallas import tpu as pltpu
from jax.experimental.pallas import tpu_sc as plsc

N = 10000
E = 320000
D = 128

NUM_TILES = 16
EPT = E // NUM_TILES
K = 80
NCH = EPT // K
NPAD = 10240
RPT = NPAD // NUM_TILES


def _sc_body(feat_hbm, ei_hbm, w_hbm, num_out, den_out,
             num_sh, den_sh, gidx_v, sidx_v, w_v, rows_v, dden_v, sem):
    c = lax.axis_index("c")
    s = lax.axis_index("s")
    zero16 = jnp.zeros((1, 16), jnp.float32)
    z16 = jnp.zeros((16,), jnp.float32)

    def zrow(i, carry):
        for g in range(D // 16):
            rows_v[pl.ds(i, 1), pl.ds(g * 16, 16)] = zero16
        return carry

    lax.fori_loop(0, K, zrow, 0)

    def zden(i, carry):
        dden_v[pl.ds(i * 16, 16)] = z16
        return carry

    lax.fori_loop(0, RPT // 16, zden, 0)

    rbase = s * RPT
    for j in range(RPT // K):
        pltpu.sync_copy(rows_v, num_sh.at[pl.ds(rbase + j * K, K)])
    pltpu.sync_copy(dden_v, den_sh.at[pl.ds(rbase, RPT)])
    plsc.subcore_barrier()

    ebase = s * EPT

    def echunk(j, carry):
        off = ebase + j * K
        pltpu.sync_copy(ei_hbm.at[pl.ds(c * E + off, K)], gidx_v)
        pltpu.sync_copy(ei_hbm.at[pl.ds((1 - c) * E + off, K)], sidx_v)
        pltpu.sync_copy(w_hbm.at[pl.ds(off, K)], w_v)
        pltpu.async_copy(feat_hbm.at[gidx_v], rows_v, sem).wait()

        def scale(g, carry2):
            w16 = w_v[pl.ds(g * 16, 16)]
            for jlane in range(16):
                k = g * 16 + jlane
                wspl2 = lax.broadcast(w16[jlane], (1, 16))
                for gg in range(D // 16):
                    rows_v[pl.ds(k, 1), pl.ds(gg * 16, 16)] = (
                        rows_v[pl.ds(k, 1), pl.ds(gg * 16, 16)] * wspl2)
            return carry2

        lax.fori_loop(0, K // 16, scale, 0)
        pltpu.sync_copy(rows_v, num_sh.at[sidx_v], add=True)
        pltpu.sync_copy(w_v, den_sh.at[sidx_v], add=True)
        return carry

    lax.fori_loop(0, NCH, echunk, 0)
    plsc.subcore_barrier()

    for j in range(RPT // K):
        base = rbase + j * K
        pltpu.sync_copy(num_sh.at[pl.ds(base, K)], rows_v)
        pltpu.sync_copy(rows_v, num_out.at[c, pl.ds(base, K)])
    pltpu.sync_copy(den_sh.at[pl.ds(rbase, RPT)], dden_v)
    pltpu.sync_copy(dden_v, den_out.at[pl.ds(c * NPAD + rbase, RPT)])


def _sc_aggregate(feat, ei, w):
    mesh = plsc.VectorSubcoreMesh(core_axis_name="c", subcore_axis_name="s")
    fn = functools.partial(
        pl.kernel,
        mesh=mesh,
        out_type=[
            jax.ShapeDtypeStruct((2, NPAD, D), jnp.float32),
            jax.ShapeDtypeStruct((2 * NPAD,), jnp.float32),
        ],
        scratch_types=[
            pltpu.VMEM_SHARED((NPAD, D), jnp.float32),
            pltpu.VMEM_SHARED((NPAD,), jnp.float32),
            pltpu.VMEM((K,), jnp.int32),
            pltpu.VMEM((K,), jnp.int32),
            pltpu.VMEM((K,), jnp.float32),
            pltpu.VMEM((K, D), jnp.float32),
            pltpu.VMEM((RPT,), jnp.float32),
            pltpu.SemaphoreType.DMA,
        ],
    )(_sc_body)
    return fn(feat, ei, w)


BN = 1000


def _tc_body(feat, n1m, d1, n2m, d2, w1, w2, wih, whh, bih, bhh, out):
    dot = functools.partial(
        lax.dot_general,
        precision=lax.Precision.HIGHEST,
        preferred_element_type=jnp.float32,
    )
    den1 = d1[...]
    den1 = jnp.where(den1 == 0.0, 1.0, den1)
    den2 = d2[...]
    den2 = jnp.where(den2 == 0.0, 1.0, den2)
    neigh1 = n1m[...] / den1
    neigh2 = n2m[...] / den2
    n1 = dot(neigh1, w1[...], (((1,), (1,)), ((), ())))
    n2 = dot(neigh2, w2[...], (((1,), (1,)), ((), ())))
    gi = (dot(n1, wih[:, :D], (((1,), (1,)), ((), ())))
          + dot(n2, wih[:, D:], (((1,), (1,)), ((), ())))
          + bih[...])
    gh = dot(feat[...], whh[...], (((1,), (1,)), ((), ()))) + bhh[...]
    i_r, i_z, i_n = gi[:, :D], gi[:, D:2 * D], gi[:, 2 * D:]
    h_r, h_z, h_n = gh[:, :D], gh[:, D:2 * D], gh[:, 2 * D:]
    r = jax.nn.sigmoid(i_r + h_r)
    z = jax.nn.sigmoid(i_z + h_z)
    nn_ = jnp.tanh(i_n + r * h_n)
    out[...] = (1.0 - z) * nn_ + z * feat[...]


def _tc_dense(feat, num1, den1, num2, den2, W1, W2, W_ih, W_hh, b_ih, b_hh):
    grid = (N // BN,)
    row_spec = pl.BlockSpec((BN, D), lambda i: (i, 0))
    den_spec = pl.BlockSpec((BN, 1), lambda i: (i, 0))

    def full(shape):
        return pl.BlockSpec(shape, lambda i: tuple(0 for _ in shape))

    return pl.pallas_call(
        _tc_body,
        grid=grid,
        in_specs=[
            row_spec, row_spec, den_spec, row_spec, den_spec,
            full((D, D)), full((D, D)), full((3 * D, 2 * D)),
            full((3 * D, D)), full((1, 3 * D)), full((1, 3 * D)),
        ],
        out_specs=row_spec,
        out_shape=jax.ShapeDtypeStruct((N, D), jnp.float32),
    )(feat, num1, den1, num2, den2, W1, W2, W_ih, W_hh,
      b_ih.reshape(1, -1), b_hh.reshape(1, -1))


def kernel(feat, edge_index, edge_weight, W1, W2, W_ih, W_hh, b_ih, b_hh):
    ei = edge_index.astype(jnp.int32).reshape(-1)
    num, den = _sc_aggregate(feat, ei, edge_weight)
    return _tc_dense(feat, num[0], den[:N, None], num[1],
                     den[NPAD:NPAD + N, None],
                     W1, W2, W_ih, W_hh, b_ih, b_hh)


# speedup vs baseline: 8.2068x; 2.0410x over previous
import functools

import jax
import jax.numpy as jnp
from jax import lax
from jax.experimental import pallas as pl
from jax.experimental.pallas import tpu as pltpu
from jax.experimental.pallas import tpu_sc as plsc

N = 10000
E = 320000
D = 128

NUM_TILES = 16
EPT = E // NUM_TILES
K = 80
NCH = EPT // K
NPAIR = NCH // 2
NPAD = 10240
RPT = NPAD // NUM_TILES


def _sc_body(feat_hbm, ei_hbm, w_hbm, num_out, den_out,
             num_sh, den_sh,
             gidx0, sidx0, w0, rows0, gidx1, sidx1, w1, rows1, dden_v,
             semi0, semi1, semg0, semg1, sems0, sems1):
    c = lax.axis_index("c")
    s = lax.axis_index("s")
    zero16 = jnp.zeros((1, 16), jnp.float32)
    z16 = jnp.zeros((16,), jnp.float32)

    def zrow(i, carry):
        for g in range(D // 16):
            rows0[pl.ds(i, 1), pl.ds(g * 16, 16)] = zero16
        return carry

    lax.fori_loop(0, K, zrow, 0)

    def zden(i, carry):
        dden_v[pl.ds(i * 16, 16)] = z16
        return carry

    lax.fori_loop(0, RPT // 16, zden, 0)

    rbase = s * RPT
    for j in range(RPT // K):
        pltpu.sync_copy(rows0, num_sh.at[pl.ds(rbase + j * K, K)])
    pltpu.sync_copy(dden_v, den_sh.at[pl.ds(rbase, RPT)])
    plsc.subcore_barrier()

    ebase = s * EPT

    def idx_cps(off, gidx, sidx, wv, sem):
        return (
            pltpu.make_async_copy(ei_hbm.at[pl.ds(c * E + off, K)], gidx, sem),
            pltpu.make_async_copy(
                ei_hbm.at[pl.ds((1 - c) * E + off, K)], sidx, sem),
            pltpu.make_async_copy(w_hbm.at[pl.ds(off, K)], wv, sem),
        )

    def issue_idx(j, gidx, sidx, wv, sem):
        off = ebase + lax.rem(j, NCH) * K
        for cp in idx_cps(off, gidx, sidx, wv, sem):
            cp.start()

    def wait_idx(gidx, sidx, wv, sem):
        for cp in idx_cps(ebase, gidx, sidx, wv, sem):
            cp.wait()

    def gather_cp(gidx, rows, sem):
        return pltpu.make_async_copy(feat_hbm.at[gidx], rows, sem)

    def scat_cps(rows, wv, sidx, sem):
        return (
            pltpu.make_async_copy(rows, num_sh.at[sidx], sem),
            pltpu.make_async_copy(wv, den_sh.at[sidx], sem),
        )

    def scale(rows, wv):
        def sgroup(g, carry):
            w16 = wv[pl.ds(g * 16, 16)]
            for jlane in range(16):
                k = g * 16 + jlane
                wspl2 = lax.broadcast(w16[jlane], (1, 16))
                for gg in range(D // 16):
                    rows[pl.ds(k, 1), pl.ds(gg * 16, 16)] = (
                        rows[pl.ds(k, 1), pl.ds(gg * 16, 16)] * wspl2)
            return carry

        lax.fori_loop(0, K // 16, sgroup, 0)

    buf0 = (gidx0, sidx0, w0, rows0, semi0, semg0, sems0)
    buf1 = (gidx1, sidx1, w1, rows1, semi1, semg1, sems1)

    def half(p, j, this, other):
        (gidx, sidx, wv, rows, semi, semg, sems) = this
        (ogidx, osidx, owv, orows, osemi, osemg, osems) = other
        wait_idx(ogidx, osidx, owv, osemi)
        gather_cp(ogidx, orows, osemg).start()
        gather_cp(gidx, rows, semg).wait()
        scale(rows, wv)
        snum, sden = scat_cps(rows, wv, sidx, sems)
        snum.start(add=True)
        sden.start(add=True)
        snum.wait()
        sden.wait()
        issue_idx(j + 2, gidx, sidx, wv, semi)

    issue_idx(0, gidx0, sidx0, w0, semi0)
    wait_idx(gidx0, sidx0, w0, semi0)
    gather_cp(gidx0, rows0, semg0).start()
    issue_idx(1, gidx1, sidx1, w1, semi1)

    def pair(p, carry):
        half(p, 2 * p, buf0, buf1)
        half(p, 2 * p + 1, buf1, buf0)
        return carry

    lax.fori_loop(0, NPAIR, pair, 0)
    wait_idx(gidx1, sidx1, w1, semi1)
    gather_cp(gidx0, rows0, semg0).wait()
    plsc.subcore_barrier()

    for j in range(RPT // K):
        base = rbase + j * K
        pltpu.sync_copy(num_sh.at[pl.ds(base, K)], rows0)
        pltpu.sync_copy(rows0, num_out.at[c, pl.ds(base, K)])
    pltpu.sync_copy(den_sh.at[pl.ds(rbase, RPT)], dden_v)
    pltpu.sync_copy(dden_v, den_out.at[pl.ds(c * NPAD + rbase, RPT)])


def _sc_aggregate(feat, ei, w):
    mesh = plsc.VectorSubcoreMesh(core_axis_name="c", subcore_axis_name="s")
    fn = functools.partial(
        pl.kernel,
        mesh=mesh,
        out_type=[
            jax.ShapeDtypeStruct((2, NPAD, D), jnp.float32),
            jax.ShapeDtypeStruct((2 * NPAD,), jnp.float32),
        ],
        scratch_types=[
            pltpu.VMEM_SHARED((NPAD, D), jnp.float32),
            pltpu.VMEM_SHARED((NPAD,), jnp.float32),
            pltpu.VMEM((K,), jnp.int32),
            pltpu.VMEM((K,), jnp.int32),
            pltpu.VMEM((K,), jnp.float32),
            pltpu.VMEM((K, D), jnp.float32),
            pltpu.VMEM((K,), jnp.int32),
            pltpu.VMEM((K,), jnp.int32),
            pltpu.VMEM((K,), jnp.float32),
            pltpu.VMEM((K, D), jnp.float32),
            pltpu.VMEM((RPT,), jnp.float32),
            pltpu.SemaphoreType.DMA,
            pltpu.SemaphoreType.DMA,
            pltpu.SemaphoreType.DMA,
            pltpu.SemaphoreType.DMA,
            pltpu.SemaphoreType.DMA,
            pltpu.SemaphoreType.DMA,
        ],
    )(_sc_body)
    return fn(feat, ei, w)


BN = 1000


def _tc_body(feat, n1m, d1, n2m, d2, w1, w2, wih, whh, bih, bhh, out):
    dot = functools.partial(
        lax.dot_general,
        precision=lax.Precision.HIGHEST,
        preferred_element_type=jnp.float32,
    )
    den1 = d1[...]
    den1 = jnp.where(den1 == 0.0, 1.0, den1)
    den2 = d2[...]
    den2 = jnp.where(den2 == 0.0, 1.0, den2)
    neigh1 = n1m[...] / den1
    neigh2 = n2m[...] / den2
    n1 = dot(neigh1, w1[...], (((1,), (1,)), ((), ())))
    n2 = dot(neigh2, w2[...], (((1,), (1,)), ((), ())))
    gi = (dot(n1, wih[:, :D], (((1,), (1,)), ((), ())))
          + dot(n2, wih[:, D:], (((1,), (1,)), ((), ())))
          + bih[...])
    gh = dot(feat[...], whh[...], (((1,), (1,)), ((), ()))) + bhh[...]
    i_r, i_z, i_n = gi[:, :D], gi[:, D:2 * D], gi[:, 2 * D:]
    h_r, h_z, h_n = gh[:, :D], gh[:, D:2 * D], gh[:, 2 * D:]
    r = jax.nn.sigmoid(i_r + h_r)
    z = jax.nn.sigmoid(i_z + h_z)
    nn_ = jnp.tanh(i_n + r * h_n)
    out[...] = (1.0 - z) * nn_ + z * feat[...]


def _tc_dense(feat, num1, den1, num2, den2, W1, W2, W_ih, W_hh, b_ih, b_hh):
    grid = (N // BN,)
    row_spec = pl.BlockSpec((BN, D), lambda i: (i, 0))
    den_spec = pl.BlockSpec((BN, 1), lambda i: (i, 0))

    def full(shape):
        return pl.BlockSpec(shape, lambda i: tuple(0 for _ in shape))

    return pl.pallas_call(
        _tc_body,
        grid=grid,
        in_specs=[
            row_spec, row_spec, den_spec, row_spec, den_spec,
            full((D, D)), full((D, D)), full((3 * D, 2 * D)),
            full((3 * D, D)), full((1, 3 * D)), full((1, 3 * D)),
        ],
        out_specs=row_spec,
        out_shape=jax.ShapeDtypeStruct((N, D), jnp.float32),
    )(feat, num1, den1, num2, den2, W1, W2, W_ih, W_hh,
      b_ih.reshape(1, -1), b_hh.reshape(1, -1))


def kernel(feat, edge_index, edge_weight, W1, W2, W_ih, W_hh, b_ih, b_hh):
    ei = edge_index.astype(jnp.int32).reshape(-1)
    num, den = _sc_aggregate(feat, ei, edge_weight)
    return _tc_dense(feat, num[0], den[:N, None], num[1],
                     den[NPAD:NPAD + N, None],
                     W1, W2, W_ih, W_hh, b_ih, b_hh)


# speedup vs baseline: 9.5294x; 1.1612x over previous
import functools

import jax
import jax.numpy as jnp
from jax import lax
from jax.experimental import pallas as pl
from jax.experimental.pallas import tpu as pltpu
from jax.experimental.pallas import tpu_sc as plsc

N = 10000
E = 320000
D = 128

NUM_TILES = 16
EPT = E // NUM_TILES
K = 80
NCH = EPT // K
NPAD = 10240
RPT = NPAD // NUM_TILES


def _sc_body(feat_hbm, ei_hbm, w_hbm, num1_out, num2_out, den1_out, den2_out,
             num_sh, den_sh,
             gidx0, gidx1, gidx2, gidx3, sidx0, sidx1, sidx2, sidx3,
             w0, w1, w2, w3, rowsa, rowsb, dden_v,
             semgi0, semgi1, semgi2, semgi3,
             semiw0, semiw1, semiw2, semiw3,
             semga, semgb, semsa, semsb):
    c = lax.axis_index("c")
    s = lax.axis_index("s")
    zero16 = jnp.zeros((1, 16), jnp.float32)
    z16 = jnp.zeros((16,), jnp.float32)
    gidx = [gidx0, gidx1, gidx2, gidx3]
    sidx = [sidx0, sidx1, sidx2, sidx3]
    wv = [w0, w1, w2, w3]
    rows = [rowsa, rowsb]
    semgi = [semgi0, semgi1, semgi2, semgi3]
    semiw = [semiw0, semiw1, semiw2, semiw3]
    semg = [semga, semgb]
    sems = [semsa, semsb]

    def zrow(i, carry):
        for g in range(D // 16):
            rowsa[pl.ds(i, 1), pl.ds(g * 16, 16)] = zero16
        return carry

    lax.fori_loop(0, K, zrow, 0)

    def zden(i, carry):
        dden_v[pl.ds(i * 16, 16)] = z16
        return carry

    lax.fori_loop(0, RPT // 16, zden, 0)

    rbase = s * RPT
    for j in range(RPT // K):
        pltpu.sync_copy(rowsa, num_sh.at[pl.ds(rbase + j * K, K)])
    pltpu.sync_copy(dden_v, den_sh.at[pl.ds(rbase, RPT)])
    plsc.subcore_barrier()

    ebase = s * EPT

    def choff(j):
        return ebase + lax.rem(j, NCH) * K

    def gidx_cp(off, q):
        return pltpu.make_async_copy(
            ei_hbm.at[pl.ds(c * E + off, K)], gidx[q], semgi[q])

    def iw_cps(off, q):
        return (
            pltpu.make_async_copy(
                ei_hbm.at[pl.ds((1 - c) * E + off, K)], sidx[q], semiw[q]),
            pltpu.make_async_copy(w_hbm.at[pl.ds(off, K)], wv[q], semiw[q]),
        )

    def gather_cp(q, b):
        return pltpu.make_async_copy(feat_hbm.at[gidx[q]], rows[b], semg[b])

    def scat_cps(q, b):
        return (
            pltpu.make_async_copy(rows[b], num_sh.at[sidx[q]], sems[b]),
            pltpu.make_async_copy(wv[q], den_sh.at[sidx[q]], sems[b]),
        )

    def scale(b, q):
        def sgroup(g, carry):
            w16 = wv[q][pl.ds(g * 16, 16)]
            for jlane in range(16):
                k = g * 16 + jlane
                wspl2 = lax.broadcast(w16[jlane], (1, 16))
                for gg in range(D // 16):
                    rows[b][pl.ds(k, 1), pl.ds(gg * 16, 16)] = (
                        rows[b][pl.ds(k, 1), pl.ds(gg * 16, 16)] * wspl2)
            return carry

        lax.fori_loop(0, K // 16, sgroup, 0)

    def chunk_step(j, q, first=False):
        b = q % 2
        nq = (q + 1) % 4
        nb = (q + 1) % 2
        gather_cp(q, b).wait()
        gidx_cp(choff(j + 4), q).start()
        if not first:
            for cp in scat_cps((q + 3) % 4, nb):
                cp.wait()
        gidx_cp(ebase, nq).wait()
        gather_cp(nq, nb).start()
        for cp in iw_cps(ebase, q):
            cp.wait()
        scale(b, q)
        snum, sden = scat_cps(q, b)
        snum.start(add=True)
        sden.start(add=True)
        iw0, iw1 = iw_cps(choff(j + 2), (q + 2) % 4)
        iw0.start()
        iw1.start()

    for q in range(4):
        gidx_cp(ebase + q * K, q).start()
    for q in range(2):
        for cp in iw_cps(ebase + q * K, q):
            cp.start()
    gidx_cp(ebase, 0).wait()
    gather_cp(0, 0).start()

    chunk_step(0, 0, first=True)
    chunk_step(1, 1)

    def quad(p, carry):
        j = 2 + 4 * p
        chunk_step(j, 2)
        chunk_step(j + 1, 3)
        chunk_step(j + 2, 0)
        chunk_step(j + 3, 1)
        return carry

    lax.fori_loop(0, (NCH - 2) // 4, quad, 0)

    for cp in scat_cps(1, 1):
        cp.wait()
    gather_cp(2, 0).wait()
    for q in (3, 0, 1):
        gidx_cp(ebase, q).wait()
    for q in (2, 3):
        for cp in iw_cps(ebase, q):
            cp.wait()
    plsc.subcore_barrier()

    @pl.when(c == 0)
    def _():
        for j in range(RPT // K):
            base = rbase + j * K
            pltpu.sync_copy(num_sh.at[pl.ds(base, K)], rowsa)
            pltpu.sync_copy(rowsa, num1_out.at[pl.ds(base, K)])
        pltpu.sync_copy(den_sh.at[pl.ds(rbase, RPT)], dden_v)
        pltpu.sync_copy(dden_v, den1_out.at[pl.ds(rbase, RPT)])

    @pl.when(c == 1)
    def _():
        for j in range(RPT // K):
            base = rbase + j * K
            pltpu.sync_copy(num_sh.at[pl.ds(base, K)], rowsb)
            pltpu.sync_copy(rowsb, num2_out.at[pl.ds(base, K)])
        pltpu.sync_copy(den_sh.at[pl.ds(rbase, RPT)], dden_v)
        pltpu.sync_copy(dden_v, den2_out.at[pl.ds(rbase, RPT)])


def _sc_aggregate(feat, ei, w):
    mesh = plsc.VectorSubcoreMesh(core_axis_name="c", subcore_axis_name="s")
    fn = functools.partial(
        pl.kernel,
        mesh=mesh,
        out_type=[
            jax.ShapeDtypeStruct((NPAD, D), jnp.float32),
            jax.ShapeDtypeStruct((NPAD, D), jnp.float32),
            jax.ShapeDtypeStruct((NPAD,), jnp.float32),
            jax.ShapeDtypeStruct((NPAD,), jnp.float32),
        ],
        scratch_types=[
            pltpu.VMEM_SHARED((NPAD, D), jnp.float32),
            pltpu.VMEM_SHARED((NPAD,), jnp.float32),
        ] + [pltpu.VMEM((K,), jnp.int32)] * 8
          + [pltpu.VMEM((K,), jnp.float32)] * 4
          + [pltpu.VMEM((K, D), jnp.float32)] * 2
          + [pltpu.VMEM((RPT,), jnp.float32)]
          + [pltpu.SemaphoreType.DMA] * 12,
    )(_sc_body)
    return fn(feat, ei, w)


BN = 1000


def _tc_body(feat, n1m, d1, n2m, d2, w1, w2, wih, whh, bih, bhh, out):
    dot = functools.partial(
        lax.dot_general,
        precision=lax.Precision.HIGHEST,
        preferred_element_type=jnp.float32,
    )
    den1 = d1[...]
    den1 = jnp.where(den1 == 0.0, 1.0, den1)
    den2 = d2[...]
    den2 = jnp.where(den2 == 0.0, 1.0, den2)
    neigh1 = n1m[...] / den1
    neigh2 = n2m[...] / den2
    n1 = dot(neigh1, w1[...], (((1,), (1,)), ((), ())))
    n2 = dot(neigh2, w2[...], (((1,), (1,)), ((), ())))
    gi = (dot(n1, wih[:, :D], (((1,), (1,)), ((), ())))
          + dot(n2, wih[:, D:], (((1,), (1,)), ((), ())))
          + bih[...])
    gh = dot(feat[...], whh[...], (((1,), (1,)), ((), ()))) + bhh[...]
    i_r, i_z, i_n = gi[:, :D], gi[:, D:2 * D], gi[:, 2 * D:]
    h_r, h_z, h_n = gh[:, :D], gh[:, D:2 * D], gh[:, 2 * D:]
    r = jax.nn.sigmoid(i_r + h_r)
    z = jax.nn.sigmoid(i_z + h_z)
    nn_ = jnp.tanh(i_n + r * h_n)
    out[...] = (1.0 - z) * nn_ + z * feat[...]


def _tc_dense(feat, num1, den1, num2, den2, W1, W2, W_ih, W_hh, b_ih, b_hh):
    grid = (N // BN,)
    row_spec = pl.BlockSpec((BN, D), lambda i: (i, 0))
    den_spec = pl.BlockSpec((BN, 1), lambda i: (i, 0))

    def full(shape):
        return pl.BlockSpec(shape, lambda i: tuple(0 for _ in shape))

    return pl.pallas_call(
        _tc_body,
        grid=grid,
        in_specs=[
            row_spec, row_spec, den_spec, row_spec, den_spec,
            full((D, D)), full((D, D)), full((3 * D, 2 * D)),
            full((3 * D, D)), full((1, 3 * D)), full((1, 3 * D)),
        ],
        out_specs=row_spec,
        out_shape=jax.ShapeDtypeStruct((N, D), jnp.float32),
    )(feat, num1, den1, num2, den2, W1, W2, W_ih, W_hh,
      b_ih.reshape(1, -1), b_hh.reshape(1, -1))


def kernel(feat, edge_index, edge_weight, W1, W2, W_ih, W_hh, b_ih, b_hh):
    ei = edge_index.astype(jnp.int32).reshape(-1)
    num1, num2, den1, den2 = _sc_aggregate(feat, ei, edge_weight)
    return _tc_dense(feat, num1, den1[:, None], num2, den2[:, None],
                     W1, W2, W_ih, W_hh, b_ih, b_hh)


# speedup vs baseline: 10.1538x; 1.0655x over previous
import functools

import jax
import jax.numpy as jnp
from jax import lax
from jax.experimental import pallas as pl
from jax.experimental.pallas import tpu as pltpu
from jax.experimental.pallas import tpu_sc as plsc

N = 10000
E = 320000
D = 128

NUM_TILES = 16
EPT = E // NUM_TILES
K = 80
NCH = EPT // K
NPAD = 10240
RPT = NPAD // NUM_TILES


def _sc_body(feat_hbm, ei_hbm, w_hbm, num1_out, num2_out, den1_out, den2_out,
             num_sh, den_sh,
             gidx0, gidx1, gidx2, gidx3, sidx0, sidx1, sidx2, sidx3,
             w0, w1, w2, w3, rowsa, rowsb, dden_v,
             semgi0, semgi1, semgi2, semgi3,
             semiw0, semiw1, semiw2, semiw3,
             semga, semgb, semsa, semsb):
    c = lax.axis_index("c")
    s = lax.axis_index("s")
    zero16 = jnp.zeros((1, 16), jnp.float32)
    z16 = jnp.zeros((16,), jnp.float32)
    gidx = [gidx0, gidx1, gidx2, gidx3]
    sidx = [sidx0, sidx1, sidx2, sidx3]
    wv = [w0, w1, w2, w3]
    rows = [rowsa, rowsb]
    semgi = [semgi0, semgi1, semgi2, semgi3]
    semiw = [semiw0, semiw1, semiw2, semiw3]
    semg = [semga, semgb]
    sems = [semsa, semsb]

    rbase = s * RPT
    ebase = s * EPT

    def choff(j):
        return ebase + lax.rem(j, NCH) * K

    def gidx_cp(off, q):
        return pltpu.make_async_copy(
            ei_hbm.at[pl.ds(c * E + off, K)], gidx[q], semgi[q])

    def iw_cps(off, q):
        return (
            pltpu.make_async_copy(
                ei_hbm.at[pl.ds((1 - c) * E + off, K)], sidx[q], semiw[q]),
            pltpu.make_async_copy(w_hbm.at[pl.ds(off, K)], wv[q], semiw[q]),
        )

    def gather_cp(q, b):
        return pltpu.make_async_copy(feat_hbm.at[gidx[q]], rows[b], semg[b])

    def scat_cps(q, b):
        return (
            pltpu.make_async_copy(rows[b], num_sh.at[sidx[q]], sems[b]),
            pltpu.make_async_copy(wv[q], den_sh.at[sidx[q]], sems[b]),
        )

    def scale(b, q):
        def sgroup(g, carry):
            w16 = wv[q][pl.ds(g * 16, 16)]
            for jlane in range(16):
                k = g * 16 + jlane
                wspl2 = lax.broadcast(w16[jlane], (1, 16))
                for gg in range(D // 16):
                    rows[b][pl.ds(k, 1), pl.ds(gg * 16, 16)] = (
                        rows[b][pl.ds(k, 1), pl.ds(gg * 16, 16)] * wspl2)
            return carry

        lax.fori_loop(0, K // 16, sgroup, 0)

    def chunk_step(j, q, first=False):
        b = q % 2
        nq = (q + 1) % 4
        nb = (q + 1) % 2
        gather_cp(q, b).wait()
        gidx_cp(choff(j + 4), q).start()
        if not first:
            for cp in scat_cps((q + 3) % 4, nb):
                cp.wait()
        gidx_cp(ebase, nq).wait()
        gather_cp(nq, nb).start()
        for cp in iw_cps(ebase, q):
            cp.wait()
        scale(b, q)
        snum, sden = scat_cps(q, b)
        snum.start(add=True)
        sden.start(add=True)
        iw0, iw1 = iw_cps(choff(j + 2), (q + 2) % 4)
        iw0.start()
        iw1.start()

    for q in range(4):
        gidx_cp(ebase + q * K, q).start()
    for q in range(2):
        for cp in iw_cps(ebase + q * K, q):
            cp.start()
    gidx_cp(ebase, 0).wait()
    gather_cp(0, 0).start()

    def zrow(i, carry):
        for g in range(D // 16):
            rowsb[pl.ds(i, 1), pl.ds(g * 16, 16)] = zero16
        return carry

    lax.fori_loop(0, K, zrow, 0)

    def zden(i, carry):
        dden_v[pl.ds(i * 16, 16)] = z16
        return carry

    lax.fori_loop(0, RPT // 16, zden, 0)

    for j in range(RPT // K):
        pltpu.sync_copy(rowsb, num_sh.at[pl.ds(rbase + j * K, K)])
    pltpu.sync_copy(dden_v, den_sh.at[pl.ds(rbase, RPT)])
    plsc.subcore_barrier()

    chunk_step(0, 0, first=True)
    chunk_step(1, 1)

    def quad(p, carry):
        j = 2 + 4 * p
        chunk_step(j, 2)
        chunk_step(j + 1, 3)
        chunk_step(j + 2, 0)
        chunk_step(j + 3, 1)
        return carry

    lax.fori_loop(0, (NCH - 2) // 4, quad, 0)

    for cp in scat_cps(1, 1):
        cp.wait()
    gather_cp(2, 0).wait()
    for q in (3, 0, 1):
        gidx_cp(ebase, q).wait()
    for q in (2, 3):
        for cp in iw_cps(ebase, q):
            cp.wait()
    plsc.subcore_barrier()

    @pl.when(c == 0)
    def _():
        for j in range(RPT // K):
            base = rbase + j * K
            pltpu.sync_copy(num_sh.at[pl.ds(base, K)], rowsa)
            pltpu.sync_copy(rowsa, num1_out.at[pl.ds(base, K)])
        pltpu.sync_copy(den_sh.at[pl.ds(rbase, RPT)], dden_v)
        pltpu.sync_copy(dden_v, den1_out.at[pl.ds(rbase, RPT)])

    @pl.when(c == 1)
    def _():
        for j in range(RPT // K):
            base = rbase + j * K
            pltpu.sync_copy(num_sh.at[pl.ds(base, K)], rowsb)
            pltpu.sync_copy(rowsb, num2_out.at[pl.ds(base, K)])
        pltpu.sync_copy(den_sh.at[pl.ds(rbase, RPT)], dden_v)
        pltpu.sync_copy(dden_v, den2_out.at[pl.ds(rbase, RPT)])


def _sc_aggregate(feat, ei, w):
    mesh = plsc.VectorSubcoreMesh(core_axis_name="c", subcore_axis_name="s")
    fn = functools.partial(
        pl.kernel,
        mesh=mesh,
        out_type=[
            jax.ShapeDtypeStruct((NPAD, D), jnp.float32),
            jax.ShapeDtypeStruct((NPAD, D), jnp.float32),
            jax.ShapeDtypeStruct((NPAD,), jnp.float32),
            jax.ShapeDtypeStruct((NPAD,), jnp.float32),
        ],
        scratch_types=[
            pltpu.VMEM_SHARED((NPAD, D), jnp.float32),
            pltpu.VMEM_SHARED((NPAD,), jnp.float32),
        ] + [pltpu.VMEM((K,), jnp.int32)] * 8
          + [pltpu.VMEM((K,), jnp.float32)] * 4
          + [pltpu.VMEM((K, D), jnp.float32)] * 2
          + [pltpu.VMEM((RPT,), jnp.float32)]
          + [pltpu.SemaphoreType.DMA] * 12,
    )(_sc_body)
    return fn(feat, ei, w)


BN = 1024


def _tc_body(feat, n1m, d1, n2m, d2, w1, w2, wih, whh, bih, bhh, out):
    dot = functools.partial(
        lax.dot_general,
        precision=lax.Precision.HIGHEST,
        preferred_element_type=jnp.float32,
    )
    den1 = d1[...]
    den1 = jnp.where(den1 == 0.0, 1.0, den1)
    den2 = d2[...]
    den2 = jnp.where(den2 == 0.0, 1.0, den2)
    neigh1 = n1m[...] / den1[:, None]
    neigh2 = n2m[...] / den2[:, None]
    n1 = dot(neigh1, w1[...], (((1,), (1,)), ((), ())))
    n2 = dot(neigh2, w2[...], (((1,), (1,)), ((), ())))
    gi = (dot(n1, wih[:, :D], (((1,), (1,)), ((), ())))
          + dot(n2, wih[:, D:], (((1,), (1,)), ((), ())))
          + bih[...])
    gh = dot(feat[...], whh[...], (((1,), (1,)), ((), ()))) + bhh[...]
    i_r, i_z, i_n = gi[:, :D], gi[:, D:2 * D], gi[:, 2 * D:]
    h_r, h_z, h_n = gh[:, :D], gh[:, D:2 * D], gh[:, 2 * D:]
    r = jax.nn.sigmoid(i_r + h_r)
    z = jax.nn.sigmoid(i_z + h_z)
    nn_ = jnp.tanh(i_n + r * h_n)
    out[...] = (1.0 - z) * nn_ + z * feat[...]


def _tc_dense(feat, num1, den1, num2, den2, W1, W2, W_ih, W_hh, b_ih, b_hh):
    grid = (pl.cdiv(N, BN),)
    row_spec = pl.BlockSpec((BN, D), lambda i: (i, 0))
    den_spec = pl.BlockSpec((BN,), lambda i: (i,))

    def full(shape):
        return pl.BlockSpec(shape, lambda i: tuple(0 for _ in shape))

    return pl.pallas_call(
        _tc_body,
        grid=grid,
        in_specs=[
            row_spec, row_spec, den_spec, row_spec, den_spec,
            full((D, D)), full((D, D)), full((3 * D, 2 * D)),
            full((3 * D, D)), full((1, 3 * D)), full((1, 3 * D)),
        ],
        out_specs=row_spec,
        out_shape=jax.ShapeDtypeStruct((N, D), jnp.float32),
    )(feat, num1, den1, num2, den2, W1, W2, W_ih, W_hh,
      b_ih.reshape(1, -1), b_hh.reshape(1, -1))


def kernel(feat, edge_index, edge_weight, W1, W2, W_ih, W_hh, b_ih, b_hh):
    ei = edge_index.astype(jnp.int32).reshape(-1)
    num1, num2, den1, den2 = _sc_aggregate(feat, ei, edge_weight)
    return _tc_dense(feat, num1, den1, num2, den2,
                     W1, W2, W_ih, W_hh, b_ih, b_hh)


# speedup vs baseline: 11.5277x; 1.1353x over previous
import functools

import jax
import jax.numpy as jnp
from jax import lax
from jax.experimental import pallas as pl
from jax.experimental.pallas import tpu as pltpu
from jax.experimental.pallas import tpu_sc as plsc

N = 10000
E = 320000
D = 128

NUM_TILES = 16
K = 128
NCHB = 156
EPTS = NCHB * K
NPAD = 10240
RPT = NPAD // NUM_TILES


def _sc_body(feat_hbm, ei_hbm, w_hbm, num1_out, num2_out, den1_out, den2_out,
             num_sh, den_sh,
             gidx0, gidx1, gidx2, gidx3, sidx0, sidx1, sidx2, sidx3,
             w0, w1, w2, w3, rowsa, rowsb, dden_v,
             semgi0, semgi1, semgi2, semgi3,
             semiw0, semiw1, semiw2, semiw3,
             semga, semgb, semsa, semsb):
    c = lax.axis_index("c")
    s = lax.axis_index("s")
    zero16 = jnp.zeros((1, 16), jnp.float32)
    z16 = jnp.zeros((16,), jnp.float32)
    gidx = [gidx0, gidx1, gidx2, gidx3]
    sidx = [sidx0, sidx1, sidx2, sidx3]
    wv = [w0, w1, w2, w3]
    rows = [rowsa, rowsb]
    semgi = [semgi0, semgi1, semgi2, semgi3]
    semiw = [semiw0, semiw1, semiw2, semiw3]
    semg = [semga, semgb]
    sems = [semsa, semsb]

    rbase = s * RPT
    ebase = s * EPTS
    ncht = jnp.where(s == NUM_TILES - 1, NCHB + 4, NCHB)

    def choff(j):
        return ebase + lax.rem(j, ncht) * K

    def gidx_cp(off, q):
        return pltpu.make_async_copy(
            ei_hbm.at[pl.ds(c * E + off, K)], gidx[q], semgi[q])

    def iw_cps(off, q):
        return (
            pltpu.make_async_copy(
                ei_hbm.at[pl.ds((1 - c) * E + off, K)], sidx[q], semiw[q]),
            pltpu.make_async_copy(w_hbm.at[pl.ds(off, K)], wv[q], semiw[q]),
        )

    def gather_cp(q, b):
        return pltpu.make_async_copy(feat_hbm.at[gidx[q]], rows[b], semg[b])

    def scat_cps(q, b):
        return (
            pltpu.make_async_copy(rows[b], num_sh.at[sidx[q]], sems[b]),
            pltpu.make_async_copy(wv[q], den_sh.at[sidx[q]], sems[b]),
        )

    def scale(b, q):
        def sgroup(g, carry):
            w16 = wv[q][pl.ds(g * 16, 16)]
            for jlane in range(16):
                k = g * 16 + jlane
                wspl2 = lax.broadcast(w16[jlane], (1, 16))
                for gg in range(D // 16):
                    rows[b][pl.ds(k, 1), pl.ds(gg * 16, 16)] = (
                        rows[b][pl.ds(k, 1), pl.ds(gg * 16, 16)] * wspl2)
            return carry

        lax.fori_loop(0, K // 16, sgroup, 0)

    def chunk_step(j, q, first=False):
        b = q % 2
        nq = (q + 1) % 4
        nb = (q + 1) % 2
        gather_cp(q, b).wait()
        gidx_cp(choff(j + 4), q).start()
        if not first:
            for cp in scat_cps((q + 3) % 4, nb):
                cp.wait()
        gidx_cp(ebase, nq).wait()
        gather_cp(nq, nb).start()
        for cp in iw_cps(ebase, q):
            cp.wait()
        scale(b, q)
        snum, sden = scat_cps(q, b)
        snum.start(add=True)
        sden.start(add=True)
        iw0, iw1 = iw_cps(choff(j + 2), (q + 2) % 4)
        iw0.start()
        iw1.start()

    for q in range(4):
        gidx_cp(ebase + q * K, q).start()
    for q in range(2):
        for cp in iw_cps(ebase + q * K, q):
            cp.start()
    gidx_cp(ebase, 0).wait()
    gather_cp(0, 0).start()

    def zrow(i, carry):
        for g in range(D // 16):
            rowsb[pl.ds(i, 1), pl.ds(g * 16, 16)] = zero16
        return carry

    lax.fori_loop(0, K, zrow, 0)

    def zden(i, carry):
        dden_v[pl.ds(i * 16, 16)] = z16
        return carry

    lax.fori_loop(0, RPT // 16, zden, 0)

    for j in range(RPT // K):
        pltpu.sync_copy(rowsb, num_sh.at[pl.ds(rbase + j * K, K)])
    pltpu.sync_copy(dden_v, den_sh.at[pl.ds(rbase, RPT)])
    plsc.subcore_barrier()

    chunk_step(0, 0, first=True)
    chunk_step(1, 1)

    def quad(p, carry):
        j = 2 + 4 * p
        chunk_step(j, 2)
        chunk_step(j + 1, 3)
        chunk_step(j + 2, 0)
        chunk_step(j + 3, 1)
        return carry

    lax.fori_loop(0, (ncht - 4) // 4, quad, 0)

    chunk_step(ncht - 2, 2)
    chunk_step(ncht - 1, 3)

    for cp in scat_cps(3, 1):
        cp.wait()
    gather_cp(0, 0).wait()
    for q in (1, 2, 3):
        gidx_cp(ebase, q).wait()
    for q in (0, 1):
        for cp in iw_cps(ebase, q):
            cp.wait()
    plsc.subcore_barrier()

    @pl.when(c == 0)
    def _():
        for j in range(RPT // K):
            base = rbase + j * K
            pltpu.sync_copy(num_sh.at[pl.ds(base, K)], rowsa)
            pltpu.sync_copy(rowsa, num1_out.at[pl.ds(base, K)])
        pltpu.sync_copy(den_sh.at[pl.ds(rbase, RPT)], dden_v)
        pltpu.sync_copy(dden_v, den1_out.at[pl.ds(rbase, RPT)])

    @pl.when(c == 1)
    def _():
        for j in range(RPT // K):
            base = rbase + j * K
            pltpu.sync_copy(num_sh.at[pl.ds(base, K)], rowsb)
            pltpu.sync_copy(rowsb, num2_out.at[pl.ds(base, K)])
        pltpu.sync_copy(den_sh.at[pl.ds(rbase, RPT)], dden_v)
        pltpu.sync_copy(dden_v, den2_out.at[pl.ds(rbase, RPT)])


def _sc_aggregate(feat, ei, w):
    mesh = plsc.VectorSubcoreMesh(core_axis_name="c", subcore_axis_name="s")
    fn = functools.partial(
        pl.kernel,
        mesh=mesh,
        out_type=[
            jax.ShapeDtypeStruct((NPAD, D), jnp.float32),
            jax.ShapeDtypeStruct((NPAD, D), jnp.float32),
            jax.ShapeDtypeStruct((NPAD,), jnp.float32),
            jax.ShapeDtypeStruct((NPAD,), jnp.float32),
        ],
        scratch_types=[
            pltpu.VMEM_SHARED((NPAD, D), jnp.float32),
            pltpu.VMEM_SHARED((NPAD,), jnp.float32),
        ] + [pltpu.VMEM((K,), jnp.int32)] * 8
          + [pltpu.VMEM((K,), jnp.float32)] * 4
          + [pltpu.VMEM((K, D), jnp.float32)] * 2
          + [pltpu.VMEM((RPT,), jnp.float32)]
          + [pltpu.SemaphoreType.DMA] * 12,
    )(_sc_body)
    return fn(feat, ei, w)


BN = 1024


def _tc_body(feat, n1m, d1, n2m, d2, w1, w2, wih, whh, bih, bhh, out):
    dot = functools.partial(
        lax.dot_general,
        precision=lax.Precision.HIGHEST,
        preferred_element_type=jnp.float32,
    )
    den1 = d1[...]
    den1 = jnp.where(den1 == 0.0, 1.0, den1)
    den2 = d2[...]
    den2 = jnp.where(den2 == 0.0, 1.0, den2)
    neigh1 = n1m[...] / den1[:, None]
    neigh2 = n2m[...] / den2[:, None]
    n1 = dot(neigh1, w1[...], (((1,), (1,)), ((), ())))
    n2 = dot(neigh2, w2[...], (((1,), (1,)), ((), ())))
    gi = (dot(n1, wih[:, :D], (((1,), (1,)), ((), ())))
          + dot(n2, wih[:, D:], (((1,), (1,)), ((), ())))
          + bih[...])
    gh = dot(feat[...], whh[...], (((1,), (1,)), ((), ()))) + bhh[...]
    i_r, i_z, i_n = gi[:, :D], gi[:, D:2 * D], gi[:, 2 * D:]
    h_r, h_z, h_n = gh[:, :D], gh[:, D:2 * D], gh[:, 2 * D:]
    r = jax.nn.sigmoid(i_r + h_r)
    z = jax.nn.sigmoid(i_z + h_z)
    nn_ = jnp.tanh(i_n + r * h_n)
    out[...] = (1.0 - z) * nn_ + z * feat[...]


def _tc_dense(feat, num1, den1, num2, den2, W1, W2, W_ih, W_hh, b_ih, b_hh):
    grid = (pl.cdiv(N, BN),)
    row_spec = pl.BlockSpec((BN, D), lambda i: (i, 0))
    den_spec = pl.BlockSpec((BN,), lambda i: (i,))

    def full(shape):
        return pl.BlockSpec(shape, lambda i: tuple(0 for _ in shape))

    return pl.pallas_call(
        _tc_body,
        grid=grid,
        in_specs=[
            row_spec, row_spec, den_spec, row_spec, den_spec,
            full((D, D)), full((D, D)), full((3 * D, 2 * D)),
            full((3 * D, D)), full((1, 3 * D)), full((1, 3 * D)),
        ],
        out_specs=row_spec,
        out_shape=jax.ShapeDtypeStruct((N, D), jnp.float32),
    )(feat, num1, den1, num2, den2, W1, W2, W_ih, W_hh,
      b_ih.reshape(1, -1), b_hh.reshape(1, -1))


def kernel(feat, edge_index, edge_weight, W1, W2, W_ih, W_hh, b_ih, b_hh):
    ei = edge_index.astype(jnp.int32).reshape(-1)
    num1, num2, den1, den2 = _sc_aggregate(feat, ei, edge_weight)
    return _tc_dense(feat, num1, den1, num2, den2,
                     W1, W2, W_ih, W_hh, b_ih, b_hh)


# speedup vs baseline: 11.6163x; 1.0077x over previous
import functools

import jax
import jax.numpy as jnp
from jax import lax
from jax.experimental import pallas as pl
from jax.experimental.pallas import tpu as pltpu
from jax.experimental.pallas import tpu_sc as plsc

N = 10000
E = 320000
D = 128

NUM_TILES = 16
K = 128
NCHB = 156
EPTS = NCHB * K
NPAD = 10240
RPT = NPAD // NUM_TILES


def _sc_body(feat_hbm, ei_hbm, w_hbm, num1_out, num2_out, den1_out, den2_out,
             num_sh, den_sh,
             gidx0, gidx1, gidx2, gidx3, sidx0, sidx1, sidx2, sidx3,
             w0, w1, w2, w3, rowsa, rowsb, dden_v,
             semgi0, semgi1, semgi2, semgi3,
             semiw0, semiw1, semiw2, semiw3,
             semga, semgb, semsa, semsb):
    c = lax.axis_index("c")
    s = lax.axis_index("s")
    zero16 = jnp.zeros((1, 16), jnp.float32)
    z16 = jnp.zeros((16,), jnp.float32)
    gidx = [gidx0, gidx1, gidx2, gidx3]
    sidx = [sidx0, sidx1, sidx2, sidx3]
    wv = [w0, w1, w2, w3]
    rows = [rowsa, rowsb]
    semgi = [semgi0, semgi1, semgi2, semgi3]
    semiw = [semiw0, semiw1, semiw2, semiw3]
    semg = [semga, semgb]
    sems = [semsa, semsb]

    rbase = s * RPT
    ebase = s * EPTS
    ncht = jnp.where(s == NUM_TILES - 1, NCHB + 4, NCHB)

    def choff(j):
        return ebase + lax.rem(j, ncht) * K

    def gidx_cp(off, q):
        return pltpu.make_async_copy(
            ei_hbm.at[pl.ds(c * E + off, K)], gidx[q], semgi[q])

    def iw_cps(off, q):
        return (
            pltpu.make_async_copy(
                ei_hbm.at[pl.ds((1 - c) * E + off, K)], sidx[q], semiw[q]),
            pltpu.make_async_copy(w_hbm.at[pl.ds(off, K)], wv[q], semiw[q]),
        )

    def gather_cp(q, b):
        return pltpu.make_async_copy(feat_hbm.at[gidx[q]], rows[b], semg[b])

    def scat_cps(q, b):
        return (
            pltpu.make_async_copy(rows[b], num_sh.at[sidx[q]], sems[b]),
            pltpu.make_async_copy(wv[q], den_sh.at[sidx[q]], sems[b]),
        )

    def scale(b, q):
        def sgroup(g, carry):
            w16 = wv[q][pl.ds(g * 16, 16)]
            for jlane in range(16):
                k = g * 16 + jlane
                wspl2 = lax.broadcast(w16[jlane], (1, 16))
                for gg in range(D // 16):
                    rows[b][pl.ds(k, 1), pl.ds(gg * 16, 16)] = (
                        rows[b][pl.ds(k, 1), pl.ds(gg * 16, 16)] * wspl2)
            return carry

        lax.fori_loop(0, K // 16, sgroup, 0)

    def chunk_step(j, q, first=False):
        b = q % 2
        nq = (q + 1) % 4
        nb = (q + 1) % 2
        gather_cp(q, b).wait()
        gidx_cp(choff(j + 4), q).start()
        if not first:
            for cp in scat_cps((q + 3) % 4, nb):
                cp.wait()
        gidx_cp(ebase, nq).wait()
        gather_cp(nq, nb).start()
        for cp in iw_cps(ebase, q):
            cp.wait()
        scale(b, q)
        snum, sden = scat_cps(q, b)
        snum.start(add=True)
        sden.start(add=True)
        iw0, iw1 = iw_cps(choff(j + 2), (q + 2) % 4)
        iw0.start()
        iw1.start()

    for q in range(4):
        gidx_cp(ebase + q * K, q).start()
    for q in range(2):
        for cp in iw_cps(ebase + q * K, q):
            cp.start()
    gidx_cp(ebase, 0).wait()
    gather_cp(0, 0).start()

    def zrow(i, carry):
        for g in range(D // 16):
            rowsb[pl.ds(i, 1), pl.ds(g * 16, 16)] = zero16
        return carry

    lax.fori_loop(0, K, zrow, 0)

    def zden(i, carry):
        dden_v[pl.ds(i * 16, 16)] = z16
        return carry

    lax.fori_loop(0, RPT // 16, zden, 0)

    for j in range(RPT // K):
        pltpu.sync_copy(rowsb, num_sh.at[pl.ds(rbase + j * K, K)])
    pltpu.sync_copy(dden_v, den_sh.at[pl.ds(rbase, RPT)])
    plsc.subcore_barrier()

    chunk_step(0, 0, first=True)
    chunk_step(1, 1)

    def quad(p, carry):
        j = 2 + 4 * p
        chunk_step(j, 2)
        chunk_step(j + 1, 3)
        chunk_step(j + 2, 0)
        chunk_step(j + 3, 1)
        return carry

    lax.fori_loop(0, (ncht - 4) // 4, quad, 0)

    chunk_step(ncht - 2, 2)
    chunk_step(ncht - 1, 3)

    for cp in scat_cps(3, 1):
        cp.wait()
    gather_cp(0, 0).wait()
    for q in (1, 2, 3):
        gidx_cp(ebase, q).wait()
    for q in (0, 1):
        for cp in iw_cps(ebase, q):
            cp.wait()
    plsc.subcore_barrier()

    def copy_out(num_out, den_out):
        nko = RPT // K
        for j in range(nko):
            base = rbase + j * K
            rb = rows[j % 2]
            if j >= 2:
                pltpu.make_async_copy(
                    rb, num_out.at[pl.ds(rbase + (j - 2) * K, K)],
                    semg[j % 2]).wait()
            pltpu.sync_copy(num_sh.at[pl.ds(base, K)], rb)
            pltpu.make_async_copy(
                rb, num_out.at[pl.ds(base, K)], semg[j % 2]).start()
        pltpu.sync_copy(den_sh.at[pl.ds(rbase, RPT)], dden_v)
        pltpu.sync_copy(dden_v, den_out.at[pl.ds(rbase, RPT)])
        for j in range(nko - 2, nko):
            pltpu.make_async_copy(
                rows[j % 2], num_out.at[pl.ds(rbase + j * K, K)],
                semg[j % 2]).wait()

    @pl.when(c == 0)
    def _():
        copy_out(num1_out, den1_out)

    @pl.when(c == 1)
    def _():
        copy_out(num2_out, den2_out)


def _sc_aggregate(feat, ei, w):
    mesh = plsc.VectorSubcoreMesh(core_axis_name="c", subcore_axis_name="s")
    fn = functools.partial(
        pl.kernel,
        mesh=mesh,
        out_type=[
            jax.ShapeDtypeStruct((NPAD, D), jnp.float32),
            jax.ShapeDtypeStruct((NPAD, D), jnp.float32),
            jax.ShapeDtypeStruct((NPAD,), jnp.float32),
            jax.ShapeDtypeStruct((NPAD,), jnp.float32),
        ],
        scratch_types=[
            pltpu.VMEM_SHARED((NPAD, D), jnp.float32),
            pltpu.VMEM_SHARED((NPAD,), jnp.float32),
        ] + [pltpu.VMEM((K,), jnp.int32)] * 8
          + [pltpu.VMEM((K,), jnp.float32)] * 4
          + [pltpu.VMEM((K, D), jnp.float32)] * 2
          + [pltpu.VMEM((RPT,), jnp.float32)]
          + [pltpu.SemaphoreType.DMA] * 12,
    )(_sc_body)
    return fn(feat, ei, w)


BN = 1024


def _tc_body(feat, n1m, d1, n2m, d2, w1, w2, wih, whh, bih, bhh, out):
    dot = functools.partial(
        lax.dot_general,
        precision=lax.Precision.HIGHEST,
        preferred_element_type=jnp.float32,
    )
    den1 = d1[...]
    den1 = jnp.where(den1 == 0.0, 1.0, den1)
    den2 = d2[...]
    den2 = jnp.where(den2 == 0.0, 1.0, den2)
    neigh1 = n1m[...] / den1[:, None]
    neigh2 = n2m[...] / den2[:, None]
    n1 = dot(neigh1, w1[...], (((1,), (1,)), ((), ())))
    n2 = dot(neigh2, w2[...], (((1,), (1,)), ((), ())))
    gi = (dot(n1, wih[:, :D], (((1,), (1,)), ((), ())))
          + dot(n2, wih[:, D:], (((1,), (1,)), ((), ())))
          + bih[...][None, :])
    gh = dot(feat[...], whh[...], (((1,), (1,)), ((), ()))) + bhh[...][None, :]
    i_r, i_z, i_n = gi[:, :D], gi[:, D:2 * D], gi[:, 2 * D:]
    h_r, h_z, h_n = gh[:, :D], gh[:, D:2 * D], gh[:, 2 * D:]
    r = jax.nn.sigmoid(i_r + h_r)
    z = jax.nn.sigmoid(i_z + h_z)
    nn_ = jnp.tanh(i_n + r * h_n)
    out[...] = (1.0 - z) * nn_ + z * feat[...]


def _tc_dense(feat, num1, den1, num2, den2, W1, W2, W_ih, W_hh, b_ih, b_hh):
    grid = (pl.cdiv(N, BN),)
    row_spec = pl.BlockSpec((BN, D), lambda i: (i, 0))
    den_spec = pl.BlockSpec((BN,), lambda i: (i,))

    def full(shape):
        return pl.BlockSpec(shape, lambda i: tuple(0 for _ in shape))

    return pl.pallas_call(
        _tc_body,
        grid=grid,
        in_specs=[
            row_spec, row_spec, den_spec, row_spec, den_spec,
            full((D, D)), full((D, D)), full((3 * D, 2 * D)),
            full((3 * D, D)), full((3 * D,)), full((3 * D,)),
        ],
        out_specs=row_spec,
        out_shape=jax.ShapeDtypeStruct((N, D), jnp.float32),
    )(feat, num1, den1, num2, den2, W1, W2, W_ih, W_hh, b_ih, b_hh)


def kernel(feat, edge_index, edge_weight, W1, W2, W_ih, W_hh, b_ih, b_hh):
    ei = edge_index.astype(jnp.int32).reshape(-1)
    num1, num2, den1, den2 = _sc_aggregate(feat, ei, edge_weight)
    return _tc_dense(feat, num1, den1, num2, den2,
                     W1, W2, W_ih, W_hh, b_ih, b_hh)
